# baseline (speedup 1.0000x reference)
import functools

import jax
import jax.numpy as jnp
from jax import lax
from jax.experimental import pallas as pl
from jax.experimental.pallas import tpu as pltpu
from jax.experimental.pallas import tpu_sc as plsc

N = 10000
E = 320000
F = 128
NW = 32
RPW = 80
ROWS_PAD = NW * RPW
EPAD = ROWS_PAD * 128
NPAD = 10240


def _leaky2(x):
    return jnp.maximum(x, 0.2 * x)


def _pre_body(hv, p, q, he, Wa, ba, We1, We2, We3, We4, be, wp, wq, whe, bal,
              lr_o, r_o, s_o, t_o, hw_o):
    hvb, pb, qb = hv[...], p[...], q[...]
    p2 = jnp.dot(pb, We2[...])
    q3 = jnp.dot(qb, We3[...])
    lr_o[...] = _leaky2(jnp.dot(hvb, Wa[...]) + ba[...])
    r_o[...] = jnp.dot(hvb, We1[...]) - p2 - q3 + be[...]
    s_o[...] = p2 + q3 + jnp.dot(hvb, We4[...])
    t_o[...] = jnp.dot(pb, wp[...]) + jnp.dot(qb, wq[...])
    hw_o[...] = jnp.dot(he[...], whe[...]) + bal[...]


def _tc_precompute(hv, p, q, he, Wa, ba, We1, We2, We3, We4, be, wp, wq, whe, bal):
    nb = 1000
    eb = 4000
    grid = (N // nb,)
    node_in = pl.BlockSpec((nb, F), lambda i: (i, 0))
    full = lambda shape: pl.BlockSpec(shape, lambda i: tuple(0 for _ in shape))
    return pl.pallas_call(
        _pre_body,
        grid=grid,
        in_specs=[
            node_in, node_in, node_in,
            pl.BlockSpec((eb, 128), lambda i: (i, 0)),
            full((F, F)), full((F,)),
            full((F, F)), full((F, F)), full((F, F)), full((F, F)), full((F,)),
            full((F, 1)), full((F, 1)), full((128, 8)), full((1,)),
        ],
        out_specs=[
            pl.BlockSpec((nb, F), lambda i: (i, 0)),
            pl.BlockSpec((nb, F), lambda i: (i, 0)),
            pl.BlockSpec((nb, F), lambda i: (i, 0)),
            pl.BlockSpec((nb, 1), lambda i: (i, 0)),
            pl.BlockSpec((eb, 8), lambda i: (i, 0)),
        ],
        out_shape=[
            jax.ShapeDtypeStruct((N, F), jnp.float32),
            jax.ShapeDtypeStruct((N, F), jnp.float32),
            jax.ShapeDtypeStruct((N, F), jnp.float32),
            jax.ShapeDtypeStruct((N, 1), jnp.float32),
            jax.ShapeDtypeStruct((E // 128 * 16, 8), jnp.float32),
        ],
    )(hv, p, q, he, Wa, ba, We1, We2, We3, We4, be, wp, wq, whe, bal)


def _sc_mv_body(recv2, send2, hw2, t_hbm, lr_hbm,
                mv_o, den_o,
                t_v, ridx8, sidx8, hw8, buf0, buf1, exrow, exrow2,
                mv_acc, den_acc, semg0, semg1):
    c = lax.axis_index("c")
    s = lax.axis_index("s")
    wid = c * 16 + s
    base = wid * RPW
    zv = jnp.zeros((16,), jnp.float32)
    zi = jnp.zeros((16,), jnp.int32)

    def _zrow(e, _):
        for v in range(8):
            buf0[e, pl.ds(v * 16, 16)] = zv
        return 0

    lax.fori_loop(0, 128, _zrow, 0)
    for v in range(8):
        exrow[0, pl.ds(v * 16, 16)] = zv
    for k in range(5):
        pltpu.sync_copy(buf0, mv_acc.at[pl.ds(s * 640 + k * 128, 128)])
        pltpu.sync_copy(exrow.at[0], den_acc.at[pl.ds(s * 640 + k * 128, 128)])
    plsc.subcore_barrier()

    pltpu.sync_copy(t_hbm, t_v.at[pl.ds(0, N)])
    pltpu.sync_copy(recv2.at[pl.ds(base, 8)], ridx8)
    pltpu.sync_copy(send2.at[pl.ds(base, 8)], sidx8)
    pltpu.sync_copy(hw2.at[pl.ds(base, 8)], hw8)
    pltpu.async_copy(lr_hbm.at[sidx8.at[0]], buf0, semg0)
    pltpu.async_copy(lr_hbm.at[sidx8.at[1]], buf1, semg1)

    def _exrow(j, dst):
        def _ex(k, _):
            si = sidx8[j, pl.ds(k * 16, 16)]
            ri = ridx8[j, pl.ds(k * 16, 16)]
            ts = plsc.load_gather(t_v, [si])
            tr = plsc.load_gather(t_v, [ri])
            hwv = hw8[j, pl.ds(k * 16, 16)]
            dst[0, pl.ds(k * 16, 16)] = jnp.exp(ts - tr + hwv)
            return 0

        lax.fori_loop(0, 8, _ex, 0)

    def _scale(buf, ex_ref):
        def _sc1(e, _):
            exb = plsc.load_gather(ex_ref, [zi, jnp.full((16,), e, jnp.int32)])
            for v in range(8):
                buf[e, pl.ds(v * 16, 16)] = buf[e, pl.ds(v * 16, 16)] * exb
            return 0

        lax.fori_loop(0, 128, _sc1, 0)

    def _body(g, _):
        ja = lax.rem(2 * g, 8)
        jb = ja + 1
        _exrow(ja, exrow)
        pltpu.sync_copy(exrow.at[0], den_acc.at[ridx8.at[ja]], add=True)
        pltpu.make_async_copy(lr_hbm.at[sidx8.at[ja]], buf0, semg0).wait()
        _scale(buf0, exrow)
        pltpu.sync_copy(buf0, mv_acc.at[ridx8.at[ja]], add=True)
        _exrow(jb, exrow2)
        pltpu.sync_copy(exrow2.at[0], den_acc.at[ridx8.at[jb]], add=True)
        pltpu.make_async_copy(lr_hbm.at[sidx8.at[jb]], buf1, semg1).wait()
        _scale(buf1, exrow2)
        pltpu.sync_copy(buf1, mv_acc.at[ridx8.at[jb]], add=True)

        @pl.when(jnp.logical_and(lax.rem(g, 4) == 3, g < RPW // 2 - 1))
        def _():
            nxt = base + lax.div(g + 1, 4) * 8
            pltpu.sync_copy(recv2.at[pl.ds(nxt, 8)], ridx8)
            pltpu.sync_copy(send2.at[pl.ds(nxt, 8)], sidx8)
            pltpu.sync_copy(hw2.at[pl.ds(nxt, 8)], hw8)

        @pl.when(g < RPW // 2 - 1)
        def _():
            jn = lax.rem(2 * g + 2, 8)
            pltpu.async_copy(lr_hbm.at[sidx8.at[jn]], buf0, semg0)
            pltpu.async_copy(lr_hbm.at[sidx8.at[jn + 1]], buf1, semg1)

        return 0

    lax.fori_loop(0, RPW // 2, _body, 0)
    plsc.subcore_barrier()

    for k in range(5):
        pltpu.sync_copy(mv_acc.at[pl.ds(s * 640 + k * 128, 128)],
                        mv_o.at[c, pl.ds(s * 640 + k * 128, 128)])
    pltpu.sync_copy(den_acc.at[pl.ds(s * 640, 640)],
                    den_o.at[c, pl.ds(s * 640, 640)])


_sc_mv = functools.partial(
    pl.kernel,
    out_type=[
        jax.ShapeDtypeStruct((2, NPAD, F), jnp.float32),
        jax.ShapeDtypeStruct((2, NPAD), jnp.float32),
    ],
    mesh=plsc.VectorSubcoreMesh(core_axis_name="c", subcore_axis_name="s",
                                num_cores=2, num_subcores=16),
    compiler_params=pltpu.CompilerParams(needs_layout_passes=False),
    scratch_types=[
        pltpu.VMEM((NPAD,), jnp.float32),
        pltpu.VMEM((8, 128), jnp.int32),
        pltpu.VMEM((8, 128), jnp.int32),
        pltpu.VMEM((8, 128), jnp.float32),
        pltpu.VMEM((128, F), jnp.float32),
        pltpu.VMEM((128, F), jnp.float32),
        pltpu.VMEM((1, 128), jnp.float32),
        pltpu.VMEM((1, 128), jnp.float32),
        pltpu.VMEM_SHARED((NPAD, F), jnp.float32),
        pltpu.VMEM_SHARED((NPAD,), jnp.float32),
        pltpu.SemaphoreType.DMA,
        pltpu.SemaphoreType.DMA,
    ],
)


def _sc_me_body(recv2, send2, r_hbm, s_hbm, me_o,
                ridx8, sidx8, ba0, bb0, ba1, bb1,
                sa0, sb0, sa1, sb1):
    c = lax.axis_index("c")
    s = lax.axis_index("s")
    wid = c * 16 + s
    base = wid * RPW

    pltpu.sync_copy(recv2.at[pl.ds(base, 8)], ridx8)
    pltpu.sync_copy(send2.at[pl.ds(base, 8)], sidx8)
    pltpu.async_copy(r_hbm.at[ridx8.at[0]], ba0, sa0)
    pltpu.async_copy(s_hbm.at[sidx8.at[0]], bb0, sb0)
    pltpu.async_copy(r_hbm.at[ridx8.at[1]], ba1, sa1)
    pltpu.async_copy(s_hbm.at[sidx8.at[1]], bb1, sb1)

    def _me(ba, bb):
        def _me1(e, _):
            for v in range(8):
                a = ba[e, pl.ds(v * 16, 16)] + bb[e, pl.ds(v * 16, 16)]
                ba[e, pl.ds(v * 16, 16)] = jnp.maximum(a, 0.2 * a)
            return 0

        lax.fori_loop(0, 128, _me1, 0)

    def _body(g, _):
        ja = lax.rem(2 * g, 8)
        pltpu.make_async_copy(r_hbm.at[ridx8.at[ja]], ba0, sa0).wait()
        pltpu.make_async_copy(s_hbm.at[sidx8.at[ja]], bb0, sb0).wait()
        _me(ba0, bb0)
        pltpu.sync_copy(ba0, me_o.at[pl.ds((base + 2 * g) * 128, 128)])
        pltpu.make_async_copy(r_hbm.at[ridx8.at[ja]], ba1, sa1).wait()
        pltpu.make_async_copy(s_hbm.at[sidx8.at[ja]], bb1, sb1).wait()
        _me(ba1, bb1)

        @pl.when(jnp.logical_and(lax.rem(g, 4) == 3, g < RPW // 2 - 1))
        def _():
            nxt = base + lax.div(g + 1, 4) * 8
            pltpu.sync_copy(recv2.at[pl.ds(nxt, 8)], ridx8)
            pltpu.sync_copy(send2.at[pl.ds(nxt, 8)], sidx8)

        pltpu.sync_copy(ba1, me_o.at[pl.ds((base + 2 * g + 1) * 128, 128)])

        @pl.when(g < RPW // 2 - 1)
        def _():
            jn = lax.rem(2 * g + 2, 8)
            pltpu.async_copy(r_hbm.at[ridx8.at[jn]], ba0, sa0)
            pltpu.async_copy(s_hbm.at[sidx8.at[jn]], bb0, sb0)
            pltpu.async_copy(r_hbm.at[ridx8.at[jn + 1]], ba1, sa1)
            pltpu.async_copy(s_hbm.at[sidx8.at[jn + 1]], bb1, sb1)

        return 0

    lax.fori_loop(0, RPW // 2, _body, 0)


_sc_me = functools.partial(
    pl.kernel,
    out_type=jax.ShapeDtypeStruct((EPAD, F), jnp.float32),
    mesh=plsc.VectorSubcoreMesh(core_axis_name="c", subcore_axis_name="s",
                                num_cores=2, num_subcores=16),
    compiler_params=pltpu.CompilerParams(needs_layout_passes=False),
    scratch_types=[
        pltpu.VMEM((8, 128), jnp.int32),
        pltpu.VMEM((8, 128), jnp.int32),
        pltpu.VMEM((128, F), jnp.float32),
        pltpu.VMEM((128, F), jnp.float32),
        pltpu.VMEM((128, F), jnp.float32),
        pltpu.VMEM((128, F), jnp.float32),
        pltpu.SemaphoreType.DMA,
        pltpu.SemaphoreType.DMA,
        pltpu.SemaphoreType.DMA,
        pltpu.SemaphoreType.DMA,
    ],
)


def _sc_edge_call(recv2, send2, hw_p, t1, lr, r_n, s_n):
    mv_part, den_part = _sc_mv(_sc_mv_body)(recv2, send2, hw_p, t1, lr)
    me_pad = _sc_me(_sc_me_body)(recv2, send2, r_n, s_n)
    return me_pad, mv_part, den_part


def _fin_body(mv2, den2, out):
    m = mv2[0] + mv2[1]
    d = den2[0, :, :1] + den2[1, :, :1]
    x = m / (d + 1e-9)
    out[...] = jnp.where(x > 0, x, jnp.exp(jnp.minimum(x, 0.0)) - 1.0)


def _tc_finalize(mv_part, den_part):
    nb = 1000
    return pl.pallas_call(
        _fin_body,
        grid=(N // nb,),
        in_specs=[
            pl.BlockSpec((2, nb, F), lambda i: (0, i, 0)),
            pl.BlockSpec((2, nb, 1), lambda i: (0, i, 0)),
        ],
        out_specs=pl.BlockSpec((nb, F), lambda i: (i, 0)),
        out_shape=jax.ShapeDtypeStruct((N, F), jnp.float32),
    )(mv_part, den_part)


def kernel(hv_ftr, he_ftr, p_ftr, q_ftr, edge_index,
           W_attend, b_attend, W_align, b_align, W_e, b_e):
    ei = edge_index.astype(jnp.int32)
    recv, send = ei[0], ei[1]

    wp, wq, whe = W_align[:F], W_align[F:2 * F], W_align[2 * F:]
    We1, We2, We3, We4 = (W_e[:F], W_e[F:2 * F], W_e[2 * F:3 * F], W_e[3 * F:])
    he2 = he_ftr.reshape(E // 8, 128)
    w16 = jnp.kron(jnp.eye(8, dtype=jnp.float32), whe)

    lr, r_n, s_n, t2, hw2 = _tc_precompute(
        hv_ftr, p_ftr, q_ftr, he2, W_attend, b_attend,
        We1, We2, We3, We4, b_e, wp, wq, w16, b_align)

    pad = EPAD - E
    zi = jnp.zeros((pad,), jnp.int32)
    recv2 = jnp.concatenate([recv, zi]).reshape(ROWS_PAD, 128)
    send2 = jnp.concatenate([send, zi]).reshape(ROWS_PAD, 128)
    hw_p = jnp.concatenate([hw2.reshape(E), jnp.full((pad,), -1e30, jnp.float32)]
                           ).reshape(ROWS_PAD, 128)

    me_pad, mv_part, den_part = _sc_edge_call(
        recv2, send2, hw_p, t2[:, 0], lr, r_n, s_n)

    mv_ftr = _tc_finalize(mv_part[:, :N], den_part[:, :N, None])
    return mv_ftr, me_pad[:E]


# speedup vs baseline: 1.1389x; 1.1389x over previous
import functools

import jax
import jax.numpy as jnp
from jax import lax
from jax.experimental import pallas as pl
from jax.experimental.pallas import tpu as pltpu
from jax.experimental.pallas import tpu_sc as plsc

N = 10000
E = 320000
F = 128
NW = 32
RPW = 80
RPW0 = 112
RPW1 = 48
ROWS_PAD = NW * RPW
EPAD = ROWS_PAD * 128
NPAD = 10240


def _leaky2(x):
    return jnp.maximum(x, 0.2 * x)


def _pre_body(hv, p, q, he, Wa, ba, We1, We2, We3, We4, be, wp, wq, whe, bal,
              lr_o, r_o, s_o, t_o, hw_o):
    hvb, pb, qb = hv[...], p[...], q[...]
    p2 = jnp.dot(pb, We2[...])
    q3 = jnp.dot(qb, We3[...])
    lr_o[...] = _leaky2(jnp.dot(hvb, Wa[...]) + ba[...])
    r_o[...] = jnp.dot(hvb, We1[...]) - p2 - q3 + be[...]
    s_o[...] = p2 + q3 + jnp.dot(hvb, We4[...])
    t_o[...] = jnp.dot(pb, wp[...]) + jnp.dot(qb, wq[...])
    hw_o[...] = jnp.dot(he[...], whe[...]) + bal[...]


def _tc_precompute(hv, p, q, he, Wa, ba, We1, We2, We3, We4, be, wp, wq, whe, bal):
    nb = 1000
    eb = 4000
    grid = (N // nb,)
    node_in = pl.BlockSpec((nb, F), lambda i: (i, 0))
    full = lambda shape: pl.BlockSpec(shape, lambda i: tuple(0 for _ in shape))
    return pl.pallas_call(
        _pre_body,
        grid=grid,
        in_specs=[
            node_in, node_in, node_in,
            pl.BlockSpec((eb, 128), lambda i: (i, 0)),
            full((F, F)), full((F,)),
            full((F, F)), full((F, F)), full((F, F)), full((F, F)), full((F,)),
            full((F, 1)), full((F, 1)), full((128, 8)), full((1,)),
        ],
        out_specs=[
            pl.BlockSpec((nb, F), lambda i: (i, 0)),
            pl.BlockSpec((nb, F), lambda i: (i, 0)),
            pl.BlockSpec((nb, F), lambda i: (i, 0)),
            pl.BlockSpec((nb, 1), lambda i: (i, 0)),
            pl.BlockSpec((eb, 8), lambda i: (i, 0)),
        ],
        out_shape=[
            jax.ShapeDtypeStruct((N, F), jnp.float32),
            jax.ShapeDtypeStruct((N, F), jnp.float32),
            jax.ShapeDtypeStruct((N, F), jnp.float32),
            jax.ShapeDtypeStruct((N, 1), jnp.float32),
            jax.ShapeDtypeStruct((E // 128 * 16, 8), jnp.float32),
        ],
    )(hv, p, q, he, Wa, ba, We1, We2, We3, We4, be, wp, wq, whe, bal)


def _sc_mv_body(recv2, send2, hw2, t_hbm, lr_hbm,
                mv_o, den_o,
                t_v, ridx8, sidx8, hw8, buf0, buf1, exrow, exrow2,
                mv_acc, den_acc, semg0, semg1):
    c = lax.axis_index("c")
    s = lax.axis_index("s")
    nrows = jnp.where(c == 0, RPW0, RPW1)
    base = pl.multiple_of(jnp.where(c == 0, s * RPW0, 16 * RPW0 + s * RPW1), 8)
    ng = nrows // 2
    zv = jnp.zeros((16,), jnp.float32)
    zi = jnp.zeros((16,), jnp.int32)

    def _zrow(e, _):
        for v in range(8):
            buf0[e, pl.ds(v * 16, 16)] = zv
        return 0

    lax.fori_loop(0, 128, _zrow, 0)
    for v in range(8):
        exrow[0, pl.ds(v * 16, 16)] = zv
    for k in range(5):
        pltpu.sync_copy(buf0, mv_acc.at[pl.ds(s * 640 + k * 128, 128)])
        pltpu.sync_copy(exrow.at[0], den_acc.at[pl.ds(s * 640 + k * 128, 128)])
    plsc.subcore_barrier()

    pltpu.sync_copy(t_hbm, t_v.at[pl.ds(0, N)])
    pltpu.sync_copy(recv2.at[pl.ds(base, 8)], ridx8)
    pltpu.sync_copy(send2.at[pl.ds(base, 8)], sidx8)
    pltpu.sync_copy(hw2.at[pl.ds(base, 8)], hw8)
    pltpu.async_copy(lr_hbm.at[sidx8.at[0]], buf0, semg0)
    pltpu.async_copy(lr_hbm.at[sidx8.at[1]], buf1, semg1)

    def _exrow(j, dst):
        def _ex(k, _):
            si = sidx8[j, pl.ds(k * 16, 16)]
            ri = ridx8[j, pl.ds(k * 16, 16)]
            ts = plsc.load_gather(t_v, [si])
            tr = plsc.load_gather(t_v, [ri])
            hwv = hw8[j, pl.ds(k * 16, 16)]
            dst[0, pl.ds(k * 16, 16)] = jnp.exp(ts - tr + hwv)
            return 0

        lax.fori_loop(0, 8, _ex, 0)

    def _scale(buf, ex_ref):
        def _sc1(e, _):
            exb = plsc.load_gather(ex_ref, [zi, jnp.full((16,), e, jnp.int32)])
            for v in range(8):
                buf[e, pl.ds(v * 16, 16)] = buf[e, pl.ds(v * 16, 16)] * exb
            return 0

        lax.fori_loop(0, 128, _sc1, 0)

    def _body(g, _):
        ja = lax.rem(2 * g, 8)
        jb = ja + 1
        _exrow(ja, exrow)
        pltpu.sync_copy(exrow.at[0], den_acc.at[ridx8.at[ja]], add=True)
        pltpu.make_async_copy(lr_hbm.at[sidx8.at[ja]], buf0, semg0).wait()
        _scale(buf0, exrow)
        pltpu.sync_copy(buf0, mv_acc.at[ridx8.at[ja]], add=True)
        _exrow(jb, exrow2)
        pltpu.sync_copy(exrow2.at[0], den_acc.at[ridx8.at[jb]], add=True)
        pltpu.make_async_copy(lr_hbm.at[sidx8.at[jb]], buf1, semg1).wait()
        _scale(buf1, exrow2)
        pltpu.sync_copy(buf1, mv_acc.at[ridx8.at[jb]], add=True)

        @pl.when(jnp.logical_and(lax.rem(g, 4) == 3, g < ng - 1))
        def _():
            nxt = base + lax.div(g + 1, 4) * 8
            pltpu.sync_copy(recv2.at[pl.ds(nxt, 8)], ridx8)
            pltpu.sync_copy(send2.at[pl.ds(nxt, 8)], sidx8)
            pltpu.sync_copy(hw2.at[pl.ds(nxt, 8)], hw8)

        @pl.when(g < ng - 1)
        def _():
            jn = lax.rem(2 * g + 2, 8)
            pltpu.async_copy(lr_hbm.at[sidx8.at[jn]], buf0, semg0)
            pltpu.async_copy(lr_hbm.at[sidx8.at[jn + 1]], buf1, semg1)

        return 0

    lax.fori_loop(0, ng, _body, 0)
    plsc.subcore_barrier()

    for k in range(5):
        pltpu.sync_copy(mv_acc.at[pl.ds(s * 640 + k * 128, 128)],
                        mv_o.at[c, pl.ds(s * 640 + k * 128, 128)])
    pltpu.sync_copy(den_acc.at[pl.ds(s * 640, 640)],
                    den_o.at[c, pl.ds(s * 640, 640)])


_sc_mv = functools.partial(
    pl.kernel,
    out_type=[
        jax.ShapeDtypeStruct((2, NPAD, F), jnp.float32),
        jax.ShapeDtypeStruct((2, NPAD), jnp.float32),
    ],
    mesh=plsc.VectorSubcoreMesh(core_axis_name="c", subcore_axis_name="s",
                                num_cores=2, num_subcores=16),
    compiler_params=pltpu.CompilerParams(needs_layout_passes=False),
    scratch_types=[
        pltpu.VMEM((NPAD,), jnp.float32),
        pltpu.VMEM((8, 128), jnp.int32),
        pltpu.VMEM((8, 128), jnp.int32),
        pltpu.VMEM((8, 128), jnp.float32),
        pltpu.VMEM((128, F), jnp.float32),
        pltpu.VMEM((128, F), jnp.float32),
        pltpu.VMEM((1, 128), jnp.float32),
        pltpu.VMEM((1, 128), jnp.float32),
        pltpu.VMEM_SHARED((NPAD, F), jnp.float32),
        pltpu.VMEM_SHARED((NPAD,), jnp.float32),
        pltpu.SemaphoreType.DMA,
        pltpu.SemaphoreType.DMA,
    ],
)


def _sc_me_body(recv2, send2, r_hbm, s_hbm, me_o,
                ridx8, sidx8, ba0, bb0, ba1, bb1,
                sa0, sb0, sa1, sb1):
    c = lax.axis_index("c")
    s = lax.axis_index("s")
    nrows = jnp.where(c == 0, RPW0, RPW1)
    base = pl.multiple_of(jnp.where(c == 0, s * RPW0, 16 * RPW0 + s * RPW1), 8)
    ng = nrows // 2

    pltpu.sync_copy(recv2.at[pl.ds(base, 8)], ridx8)
    pltpu.sync_copy(send2.at[pl.ds(base, 8)], sidx8)
    pltpu.async_copy(r_hbm.at[ridx8.at[0]], ba0, sa0)
    pltpu.async_copy(s_hbm.at[sidx8.at[0]], bb0, sb0)
    pltpu.async_copy(r_hbm.at[ridx8.at[1]], ba1, sa1)
    pltpu.async_copy(s_hbm.at[sidx8.at[1]], bb1, sb1)

    def _me(ba, bb):
        def _me1(e, _):
            for v in range(8):
                a = ba[e, pl.ds(v * 16, 16)] + bb[e, pl.ds(v * 16, 16)]
                ba[e, pl.ds(v * 16, 16)] = jnp.maximum(a, 0.2 * a)
            return 0

        lax.fori_loop(0, 128, _me1, 0)

    def _body(g, _):
        ja = lax.rem(2 * g, 8)
        pltpu.make_async_copy(r_hbm.at[ridx8.at[ja]], ba0, sa0).wait()
        pltpu.make_async_copy(s_hbm.at[sidx8.at[ja]], bb0, sb0).wait()
        _me(ba0, bb0)
        pltpu.sync_copy(ba0, me_o.at[pl.ds((base + 2 * g) * 128, 128)])
        pltpu.make_async_copy(r_hbm.at[ridx8.at[ja]], ba1, sa1).wait()
        pltpu.make_async_copy(s_hbm.at[sidx8.at[ja]], bb1, sb1).wait()
        _me(ba1, bb1)

        @pl.when(jnp.logical_and(lax.rem(g, 4) == 3, g < ng - 1))
        def _():
            nxt = base + lax.div(g + 1, 4) * 8
            pltpu.sync_copy(recv2.at[pl.ds(nxt, 8)], ridx8)
            pltpu.sync_copy(send2.at[pl.ds(nxt, 8)], sidx8)

        pltpu.sync_copy(ba1, me_o.at[pl.ds((base + 2 * g + 1) * 128, 128)])

        @pl.when(g < ng - 1)
        def _():
            jn = lax.rem(2 * g + 2, 8)
            pltpu.async_copy(r_hbm.at[ridx8.at[jn]], ba0, sa0)
            pltpu.async_copy(s_hbm.at[sidx8.at[jn]], bb0, sb0)
            pltpu.async_copy(r_hbm.at[ridx8.at[jn + 1]], ba1, sa1)
            pltpu.async_copy(s_hbm.at[sidx8.at[jn + 1]], bb1, sb1)

        return 0

    lax.fori_loop(0, ng, _body, 0)


_sc_me = functools.partial(
    pl.kernel,
    out_type=jax.ShapeDtypeStruct((EPAD, F), jnp.float32),
    mesh=plsc.VectorSubcoreMesh(core_axis_name="c", subcore_axis_name="s",
                                num_cores=2, num_subcores=16),
    compiler_params=pltpu.CompilerParams(needs_layout_passes=False),
    scratch_types=[
        pltpu.VMEM((8, 128), jnp.int32),
        pltpu.VMEM((8, 128), jnp.int32),
        pltpu.VMEM((128, F), jnp.float32),
        pltpu.VMEM((128, F), jnp.float32),
        pltpu.VMEM((128, F), jnp.float32),
        pltpu.VMEM((128, F), jnp.float32),
        pltpu.SemaphoreType.DMA,
        pltpu.SemaphoreType.DMA,
        pltpu.SemaphoreType.DMA,
        pltpu.SemaphoreType.DMA,
    ],
)


def _sc_edge_call(recv2, send2, hw_p, t1, lr, r_n, s_n):
    mv_part, den_part = _sc_mv(_sc_mv_body)(recv2, send2, hw_p, t1, lr)
    me_pad = _sc_me(_sc_me_body)(recv2, send2, r_n, s_n)
    return me_pad, mv_part, den_part


def _fin_body(mv2, den2, out):
    m = mv2[0] + mv2[1]
    d = den2[0, :, :1] + den2[1, :, :1]
    x = m / (d + 1e-9)
    out[...] = jnp.where(x > 0, x, jnp.exp(jnp.minimum(x, 0.0)) - 1.0)


def _tc_finalize(mv_part, den_part):
    nb = 1000
    return pl.pallas_call(
        _fin_body,
        grid=(N // nb,),
        in_specs=[
            pl.BlockSpec((2, nb, F), lambda i: (0, i, 0)),
            pl.BlockSpec((2, nb, 1), lambda i: (0, i, 0)),
        ],
        out_specs=pl.BlockSpec((nb, F), lambda i: (i, 0)),
        out_shape=jax.ShapeDtypeStruct((N, F), jnp.float32),
    )(mv_part, den_part)


def kernel(hv_ftr, he_ftr, p_ftr, q_ftr, edge_index,
           W_attend, b_attend, W_align, b_align, W_e, b_e):
    ei = edge_index.astype(jnp.int32)
    recv, send = ei[0], ei[1]

    wp, wq, whe = W_align[:F], W_align[F:2 * F], W_align[2 * F:]
    We1, We2, We3, We4 = (W_e[:F], W_e[F:2 * F], W_e[2 * F:3 * F], W_e[3 * F:])
    he2 = he_ftr.reshape(E // 8, 128)
    w16 = jnp.kron(jnp.eye(8, dtype=jnp.float32), whe)

    lr, r_n, s_n, t2, hw2 = _tc_precompute(
        hv_ftr, p_ftr, q_ftr, he2, W_attend, b_attend,
        We1, We2, We3, We4, b_e, wp, wq, w16, b_align)

    pad = EPAD - E
    zi = jnp.zeros((pad,), jnp.int32)
    recv2 = jnp.concatenate([recv, zi]).reshape(ROWS_PAD, 128)
    send2 = jnp.concatenate([send, zi]).reshape(ROWS_PAD, 128)
    hw_p = jnp.concatenate([hw2.reshape(E), jnp.full((pad,), -1e30, jnp.float32)]
                           ).reshape(ROWS_PAD, 128)

    me_pad, mv_part, den_part = _sc_edge_call(
        recv2, send2, hw_p, t2[:, 0], lr, r_n, s_n)

    mv_ftr = _tc_finalize(mv_part[:, :N], den_part[:, :N, None])
    return mv_ftr, me_pad[:E]


# speedup vs baseline: 1.9973x; 1.7538x over previous
import functools

import jax
import jax.numpy as jnp
from jax import lax
from jax.experimental import pallas as pl
from jax.experimental.pallas import tpu as pltpu
from jax.experimental.pallas import tpu_sc as plsc

N = 10000
E = 320000
F = 128
NW = 32
RPW = 80
RPW0 = 80
RPW1 = 80
ROWS_PAD = NW * RPW
EPAD = ROWS_PAD * 128
NPAD = 10240


def _leaky2(x):
    return jnp.maximum(x, 0.2 * x)


def _pre_body(hv, p, q, he, Wa, ba, We1, We2, We3, We4, be, wp, wq, whe, bal,
              lr_o, r_o, s_o, t_o, hw_o):
    hvb, pb, qb = hv[...], p[...], q[...]
    p2 = jnp.dot(pb, We2[...])
    q3 = jnp.dot(qb, We3[...])
    lr_o[...] = _leaky2(jnp.dot(hvb, Wa[...]) + ba[...])
    r_o[...] = jnp.dot(hvb, We1[...]) - p2 - q3 + be[...]
    s_o[...] = p2 + q3 + jnp.dot(hvb, We4[...])
    t_o[...] = jnp.dot(pb, wp[...]) + jnp.dot(qb, wq[...])
    hw_o[...] = jnp.dot(he[...], whe[...]) + bal[...]


def _tc_precompute(hv, p, q, he, Wa, ba, We1, We2, We3, We4, be, wp, wq, whe, bal):
    nb = 1000
    eb = 4000
    grid = (N // nb,)
    node_in = pl.BlockSpec((nb, F), lambda i: (i, 0))
    full = lambda shape: pl.BlockSpec(shape, lambda i: tuple(0 for _ in shape))
    return pl.pallas_call(
        _pre_body,
        grid=grid,
        in_specs=[
            node_in, node_in, node_in,
            pl.BlockSpec((eb, 128), lambda i: (i, 0)),
            full((F, F)), full((F,)),
            full((F, F)), full((F, F)), full((F, F)), full((F, F)), full((F,)),
            full((F, 1)), full((F, 1)), full((128, 8)), full((1,)),
        ],
        out_specs=[
            pl.BlockSpec((nb, F), lambda i: (i, 0)),
            pl.BlockSpec((nb, F), lambda i: (i, 0)),
            pl.BlockSpec((nb, F), lambda i: (i, 0)),
            pl.BlockSpec((nb, 1), lambda i: (i, 0)),
            pl.BlockSpec((eb, 8), lambda i: (i, 0)),
        ],
        out_shape=[
            jax.ShapeDtypeStruct((N, F), jnp.float32),
            jax.ShapeDtypeStruct((N, F), jnp.float32),
            jax.ShapeDtypeStruct((N, F), jnp.float32),
            jax.ShapeDtypeStruct((N, 1), jnp.float32),
            jax.ShapeDtypeStruct((E // 128 * 16, 8), jnp.float32),
        ],
    )(hv, p, q, he, Wa, ba, We1, We2, We3, We4, be, wp, wq, whe, bal)


def _sc_mv_body(recv2, send2, hw2, t_hbm, lr_hbm,
                mv_o, den_o,
                t_v, ridx8, sidx8, hw8, buf0, buf1, exrow, exrow2,
                mv_acc, den_acc, semg0, semg1):
    c = lax.axis_index("c")
    s = lax.axis_index("s")
    nrows = jnp.where(c == 0, RPW0, RPW1)
    base = pl.multiple_of(jnp.where(c == 0, s * RPW0, 16 * RPW0 + s * RPW1), 8)
    ng = nrows // 2
    zv = jnp.zeros((16,), jnp.float32)
    zi = jnp.zeros((16,), jnp.int32)

    def _zrow(e, _):
        for v in range(8):
            buf0[e, pl.ds(v * 16, 16)] = zv
        return 0

    lax.fori_loop(0, 128, _zrow, 0)
    for v in range(8):
        exrow[0, pl.ds(v * 16, 16)] = zv
    for k in range(5):
        pltpu.sync_copy(buf0, mv_acc.at[pl.ds(s * 640 + k * 128, 128)])
        pltpu.sync_copy(exrow.at[0], den_acc.at[pl.ds(s * 640 + k * 128, 128)])
    plsc.subcore_barrier()

    pltpu.sync_copy(t_hbm, t_v.at[pl.ds(0, N)])
    pltpu.sync_copy(recv2.at[pl.ds(base, 8)], ridx8)
    pltpu.sync_copy(send2.at[pl.ds(base, 8)], sidx8)
    pltpu.sync_copy(hw2.at[pl.ds(base, 8)], hw8)
    pltpu.async_copy(lr_hbm.at[sidx8.at[0]], buf0, semg0)
    pltpu.async_copy(lr_hbm.at[sidx8.at[1]], buf1, semg1)

    def _exrow(j, dst):
        def _ex(k, _):
            si = sidx8[j, pl.ds(k * 16, 16)]
            ri = ridx8[j, pl.ds(k * 16, 16)]
            ts = plsc.load_gather(t_v, [si])
            tr = plsc.load_gather(t_v, [ri])
            hwv = hw8[j, pl.ds(k * 16, 16)]
            dst[0, pl.ds(k * 16, 16)] = jnp.exp(ts - tr + hwv)
            return 0

        lax.fori_loop(0, 8, _ex, 0)

    def _scale(buf, ex_ref):
        def _sc1(e, _):
            exb = plsc.load_gather(ex_ref, [zi, jnp.full((16,), e, jnp.int32)])
            for v in range(8):
                buf[e, pl.ds(v * 16, 16)] = buf[e, pl.ds(v * 16, 16)] * exb
            return 0

        lax.fori_loop(0, 128, _sc1, 0)

    def _body(g, _):
        ja = lax.rem(2 * g, 8)
        jb = ja + 1
        _exrow(ja, exrow)
        pltpu.sync_copy(exrow.at[0], den_acc.at[ridx8.at[ja]], add=True)
        pltpu.make_async_copy(lr_hbm.at[sidx8.at[ja]], buf0, semg0).wait()
        _scale(buf0, exrow)
        pltpu.sync_copy(buf0, mv_acc.at[ridx8.at[ja]], add=True)
        _exrow(jb, exrow2)
        pltpu.sync_copy(exrow2.at[0], den_acc.at[ridx8.at[jb]], add=True)
        pltpu.make_async_copy(lr_hbm.at[sidx8.at[jb]], buf1, semg1).wait()
        _scale(buf1, exrow2)
        pltpu.sync_copy(buf1, mv_acc.at[ridx8.at[jb]], add=True)

        @pl.when(jnp.logical_and(lax.rem(g, 4) == 3, g < ng - 1))
        def _():
            nxt = base + lax.div(g + 1, 4) * 8
            pltpu.sync_copy(recv2.at[pl.ds(nxt, 8)], ridx8)
            pltpu.sync_copy(send2.at[pl.ds(nxt, 8)], sidx8)
            pltpu.sync_copy(hw2.at[pl.ds(nxt, 8)], hw8)

        @pl.when(g < ng - 1)
        def _():
            jn = lax.rem(2 * g + 2, 8)
            pltpu.async_copy(lr_hbm.at[sidx8.at[jn]], buf0, semg0)
            pltpu.async_copy(lr_hbm.at[sidx8.at[jn + 1]], buf1, semg1)

        return 0

    lax.fori_loop(0, ng, _body, 0)
    plsc.subcore_barrier()

    for k in range(5):
        pltpu.sync_copy(mv_acc.at[pl.ds(s * 640 + k * 128, 128)],
                        mv_o.at[c, pl.ds(s * 640 + k * 128, 128)])
    pltpu.sync_copy(den_acc.at[pl.ds(s * 640, 640)],
                    den_o.at[c, pl.ds(s * 640, 640)])


_sc_mv = functools.partial(
    pl.kernel,
    out_type=[
        jax.ShapeDtypeStruct((2, NPAD, F), jnp.float32),
        jax.ShapeDtypeStruct((2, NPAD), jnp.float32),
    ],
    mesh=plsc.VectorSubcoreMesh(core_axis_name="c", subcore_axis_name="s",
                                num_cores=2, num_subcores=16),
    compiler_params=pltpu.CompilerParams(needs_layout_passes=False),
    scratch_types=[
        pltpu.VMEM((NPAD,), jnp.float32),
        pltpu.VMEM((8, 128), jnp.int32),
        pltpu.VMEM((8, 128), jnp.int32),
        pltpu.VMEM((8, 128), jnp.float32),
        pltpu.VMEM((128, F), jnp.float32),
        pltpu.VMEM((128, F), jnp.float32),
        pltpu.VMEM((1, 128), jnp.float32),
        pltpu.VMEM((1, 128), jnp.float32),
        pltpu.VMEM_SHARED((NPAD, F), jnp.float32),
        pltpu.VMEM_SHARED((NPAD,), jnp.float32),
        pltpu.SemaphoreType.DMA,
        pltpu.SemaphoreType.DMA,
    ],
)


def _sc_me_body(recv2, send2, r_hbm, s_hbm, me_o,
                ridx8, sidx8, ba0, bb0, ba1, bb1,
                sa0, sb0, sa1, sb1):
    c = lax.axis_index("c")
    s = lax.axis_index("s")
    nrows = jnp.where(c == 0, RPW0, RPW1)
    base = pl.multiple_of(jnp.where(c == 0, s * RPW0, 16 * RPW0 + s * RPW1), 8)
    ng = nrows // 2

    pltpu.sync_copy(recv2.at[pl.ds(base, 8)], ridx8)
    pltpu.sync_copy(send2.at[pl.ds(base, 8)], sidx8)
    pltpu.async_copy(r_hbm.at[ridx8.at[0]], ba0, sa0)
    pltpu.async_copy(s_hbm.at[sidx8.at[0]], bb0, sb0)
    pltpu.async_copy(r_hbm.at[ridx8.at[1]], ba1, sa1)
    pltpu.async_copy(s_hbm.at[sidx8.at[1]], bb1, sb1)

    def _me(ba, bb):
        def _me1(e, _):
            for v in range(8):
                a = ba[e, pl.ds(v * 16, 16)] + bb[e, pl.ds(v * 16, 16)]
                ba[e, pl.ds(v * 16, 16)] = jnp.maximum(a, 0.2 * a)
            return 0

        lax.fori_loop(0, 128, _me1, 0)

    def _body(g, _):
        ja = lax.rem(2 * g, 8)
        pltpu.make_async_copy(r_hbm.at[ridx8.at[ja]], ba0, sa0).wait()
        pltpu.make_async_copy(s_hbm.at[sidx8.at[ja]], bb0, sb0).wait()
        _me(ba0, bb0)
        pltpu.sync_copy(ba0, me_o.at[pl.ds((base + 2 * g) * 128, 128)])
        pltpu.make_async_copy(r_hbm.at[ridx8.at[ja]], ba1, sa1).wait()
        pltpu.make_async_copy(s_hbm.at[sidx8.at[ja]], bb1, sb1).wait()
        _me(ba1, bb1)

        @pl.when(jnp.logical_and(lax.rem(g, 4) == 3, g < ng - 1))
        def _():
            nxt = base + lax.div(g + 1, 4) * 8
            pltpu.sync_copy(recv2.at[pl.ds(nxt, 8)], ridx8)
            pltpu.sync_copy(send2.at[pl.ds(nxt, 8)], sidx8)

        pltpu.sync_copy(ba1, me_o.at[pl.ds((base + 2 * g + 1) * 128, 128)])

        @pl.when(g < ng - 1)
        def _():
            jn = lax.rem(2 * g + 2, 8)
            pltpu.async_copy(r_hbm.at[ridx8.at[jn]], ba0, sa0)
            pltpu.async_copy(s_hbm.at[sidx8.at[jn]], bb0, sb0)
            pltpu.async_copy(r_hbm.at[ridx8.at[jn + 1]], ba1, sa1)
            pltpu.async_copy(s_hbm.at[sidx8.at[jn + 1]], bb1, sb1)

        return 0

    lax.fori_loop(0, ng, _body, 0)


_sc_me = functools.partial(
    pl.kernel,
    out_type=jax.ShapeDtypeStruct((EPAD, F), jnp.float32),
    mesh=plsc.VectorSubcoreMesh(core_axis_name="c", subcore_axis_name="s",
                                num_cores=2, num_subcores=16),
    compiler_params=pltpu.CompilerParams(needs_layout_passes=False),
    scratch_types=[
        pltpu.VMEM((8, 128), jnp.int32),
        pltpu.VMEM((8, 128), jnp.int32),
        pltpu.VMEM((128, F), jnp.float32),
        pltpu.VMEM((128, F), jnp.float32),
        pltpu.VMEM((128, F), jnp.float32),
        pltpu.VMEM((128, F), jnp.float32),
        pltpu.SemaphoreType.DMA,
        pltpu.SemaphoreType.DMA,
        pltpu.SemaphoreType.DMA,
        pltpu.SemaphoreType.DMA,
    ],
)


def _sc_edge_call(recv2, send2, hw_p, t1, lr, r_n, s_n):
    mv_part, den_part = _sc_mv(_sc_mv_body)(recv2, send2, hw_p, t1, lr)
    me_pad = _sc_me(_sc_me_body)(recv2, send2, r_n, s_n)
    return me_pad, mv_part, den_part


def _fin_body(mv2, den2, out):
    m = mv2[0] + mv2[1]
    d = den2[0, :, :1] + den2[1, :, :1]
    x = m / (d + 1e-9)
    out[...] = jnp.where(x > 0, x, jnp.exp(jnp.minimum(x, 0.0)) - 1.0)


def _tc_finalize(mv_part, den_part):
    nb = 1000
    return pl.pallas_call(
        _fin_body,
        grid=(N // nb,),
        in_specs=[
            pl.BlockSpec((2, nb, F), lambda i: (0, i, 0)),
            pl.BlockSpec((2, nb, 1), lambda i: (0, i, 0)),
        ],
        out_specs=pl.BlockSpec((nb, F), lambda i: (i, 0)),
        out_shape=jax.ShapeDtypeStruct((N, F), jnp.float32),
    )(mv_part, den_part)


def kernel(hv_ftr, he_ftr, p_ftr, q_ftr, edge_index,
           W_attend, b_attend, W_align, b_align, W_e, b_e):
    ei = edge_index.astype(jnp.int32)
    recv, send = ei[0], ei[1]

    wp, wq, whe = W_align[:F], W_align[F:2 * F], W_align[2 * F:]
    We1, We2, We3, We4 = (W_e[:F], W_e[F:2 * F], W_e[2 * F:3 * F], W_e[3 * F:])
    he2 = he_ftr.reshape(E // 8, 128)
    w16 = jnp.kron(jnp.eye(8, dtype=jnp.float32), whe)

    lr, r_n, s_n, t2, hw2 = _tc_precompute(
        hv_ftr, p_ftr, q_ftr, he2, W_attend, b_attend,
        We1, We2, We3, We4, b_e, wp, wq, w16, b_align)

    pad = EPAD - E
    zi = (jnp.arange(pad, dtype=jnp.int32) * 131) % N
    recv2 = jnp.concatenate([recv, zi]).reshape(ROWS_PAD, 128)
    send2 = jnp.concatenate([send, zi]).reshape(ROWS_PAD, 128)
    hw_p = jnp.concatenate([hw2.reshape(E), jnp.full((pad,), -1e30, jnp.float32)]
                           ).reshape(ROWS_PAD, 128)

    me_pad, mv_part, den_part = _sc_edge_call(
        recv2, send2, hw_p, t2[:, 0], lr, r_n, s_n)

    mv_ftr = _tc_finalize(mv_part[:, :N], den_part[:, :N, None])
    return mv_ftr, me_pad[:E]


# speedup vs baseline: 2.1266x; 1.0647x over previous
import functools

import jax
import jax.numpy as jnp
from jax import lax
from jax.experimental import pallas as pl
from jax.experimental.pallas import tpu as pltpu
from jax.experimental.pallas import tpu_sc as plsc

N = 10000
E = 320000
F = 128
NW = 32
RPW = 80
RPW0 = 80
RPW1 = 80
ROWS_PAD = NW * RPW
EPAD = ROWS_PAD * 128
NPAD = 10240


def _leaky2(x):
    return jnp.maximum(x, 0.2 * x)


def _pre_body(hv, p, q, he, Wa, ba, We1, We2, We3, We4, be, wp, wq, whe, bal,
              lr_o, r_o, s_o, t_o, hw_o):
    hvb, pb, qb = hv[...], p[...], q[...]
    p2 = jnp.dot(pb, We2[...])
    q3 = jnp.dot(qb, We3[...])
    lr_o[...] = _leaky2(jnp.dot(hvb, Wa[...]) + ba[...])
    r_o[...] = jnp.dot(hvb, We1[...]) - p2 - q3 + be[...]
    s_o[...] = p2 + q3 + jnp.dot(hvb, We4[...])
    t_o[...] = jnp.dot(pb, wp[...]) + jnp.dot(qb, wq[...])
    hw_o[...] = jnp.dot(he[...], whe[...]) + bal[...]


def _tc_precompute(hv, p, q, he, Wa, ba, We1, We2, We3, We4, be, wp, wq, whe, bal):
    nb = 1000
    eb = 4000
    grid = (N // nb,)
    node_in = pl.BlockSpec((nb, F), lambda i: (i, 0))
    full = lambda shape: pl.BlockSpec(shape, lambda i: tuple(0 for _ in shape))
    return pl.pallas_call(
        _pre_body,
        grid=grid,
        in_specs=[
            node_in, node_in, node_in,
            pl.BlockSpec((eb, 128), lambda i: (i, 0)),
            full((F, F)), full((F,)),
            full((F, F)), full((F, F)), full((F, F)), full((F, F)), full((F,)),
            full((F, 1)), full((F, 1)), full((128, 8)), full((1,)),
        ],
        out_specs=[
            pl.BlockSpec((nb, F), lambda i: (i, 0)),
            pl.BlockSpec((nb, F), lambda i: (i, 0)),
            pl.BlockSpec((nb, F), lambda i: (i, 0)),
            pl.BlockSpec((nb, 1), lambda i: (i, 0)),
            pl.BlockSpec((eb, 8), lambda i: (i, 0)),
        ],
        out_shape=[
            jax.ShapeDtypeStruct((N, F), jnp.float32),
            jax.ShapeDtypeStruct((N, F), jnp.float32),
            jax.ShapeDtypeStruct((N, F), jnp.float32),
            jax.ShapeDtypeStruct((N, 1), jnp.float32),
            jax.ShapeDtypeStruct((E // 128 * 16, 8), jnp.float32),
        ],
    )(hv, p, q, he, Wa, ba, We1, We2, We3, We4, be, wp, wq, whe, bal)


def _sc_mv_body(recv2, send2, hw2, t_hbm, lr_hbm,
                mv_o, den_o,
                t_v, ridx8, sidx8, hw8, buf0, buf1, exrow, exrow2,
                mv_acc, den_acc, semg0, semg1):
    c = lax.axis_index("c")
    s = lax.axis_index("s")
    nrows = jnp.where(c == 0, RPW0, RPW1)
    base = pl.multiple_of(jnp.where(c == 0, s * RPW0, 16 * RPW0 + s * RPW1), 8)
    ng = nrows // 2
    zv = jnp.zeros((16,), jnp.float32)
    zi = jnp.zeros((16,), jnp.int32)

    def _zrow(e, _):
        for v in range(8):
            buf0[e, pl.ds(v * 16, 16)] = zv
        return 0

    lax.fori_loop(0, 128, _zrow, 0)
    for v in range(8):
        exrow[0, pl.ds(v * 16, 16)] = zv
    for k in range(5):
        pltpu.sync_copy(buf0, mv_acc.at[pl.ds(s * 640 + k * 128, 128)])
        pltpu.sync_copy(exrow.at[0], den_acc.at[pl.ds(s * 640 + k * 128, 128)])
    plsc.subcore_barrier()

    pltpu.sync_copy(t_hbm, t_v.at[pl.ds(0, N)])
    pltpu.sync_copy(recv2.at[pl.ds(base, 8)], ridx8)
    pltpu.sync_copy(send2.at[pl.ds(base, 8)], sidx8)
    pltpu.sync_copy(hw2.at[pl.ds(base, 8)], hw8)
    pltpu.async_copy(lr_hbm.at[sidx8.at[0]], buf0, semg0)
    pltpu.async_copy(lr_hbm.at[sidx8.at[1]], buf1, semg1)

    def _exrow(j, dst):
        @plsc.parallel_loop(0, 8, unroll=2)
        def _ex(k):
            si = sidx8[j, pl.ds(k * 16, 16)]
            ri = ridx8[j, pl.ds(k * 16, 16)]
            ts = plsc.load_gather(t_v, [si])
            tr = plsc.load_gather(t_v, [ri])
            hwv = hw8[j, pl.ds(k * 16, 16)]
            dst[0, pl.ds(k * 16, 16)] = jnp.exp(ts - tr + hwv)

    def _scale(buf, ex_ref):
        @plsc.parallel_loop(0, 128, unroll=4)
        def _sc1(e):
            exb = plsc.load_gather(ex_ref, [zi, jnp.full((16,), e, jnp.int32)])
            for v in range(8):
                buf[e, pl.ds(v * 16, 16)] = buf[e, pl.ds(v * 16, 16)] * exb

    def _body(g, _):
        ja = lax.rem(2 * g, 8)
        jb = ja + 1
        _exrow(ja, exrow)
        pltpu.sync_copy(exrow.at[0], den_acc.at[ridx8.at[ja]], add=True)
        pltpu.make_async_copy(lr_hbm.at[sidx8.at[ja]], buf0, semg0).wait()
        _scale(buf0, exrow)
        pltpu.sync_copy(buf0, mv_acc.at[ridx8.at[ja]], add=True)
        _exrow(jb, exrow2)
        pltpu.sync_copy(exrow2.at[0], den_acc.at[ridx8.at[jb]], add=True)
        pltpu.make_async_copy(lr_hbm.at[sidx8.at[jb]], buf1, semg1).wait()
        _scale(buf1, exrow2)
        pltpu.sync_copy(buf1, mv_acc.at[ridx8.at[jb]], add=True)

        @pl.when(jnp.logical_and(lax.rem(g, 4) == 3, g < ng - 1))
        def _():
            nxt = base + lax.div(g + 1, 4) * 8
            pltpu.sync_copy(recv2.at[pl.ds(nxt, 8)], ridx8)
            pltpu.sync_copy(send2.at[pl.ds(nxt, 8)], sidx8)
            pltpu.sync_copy(hw2.at[pl.ds(nxt, 8)], hw8)

        @pl.when(g < ng - 1)
        def _():
            jn = lax.rem(2 * g + 2, 8)
            pltpu.async_copy(lr_hbm.at[sidx8.at[jn]], buf0, semg0)
            pltpu.async_copy(lr_hbm.at[sidx8.at[jn + 1]], buf1, semg1)

        return 0

    lax.fori_loop(0, ng, _body, 0)
    plsc.subcore_barrier()

    for k in range(5):
        pltpu.sync_copy(mv_acc.at[pl.ds(s * 640 + k * 128, 128)],
                        mv_o.at[c, pl.ds(s * 640 + k * 128, 128)])
    pltpu.sync_copy(den_acc.at[pl.ds(s * 640, 640)],
                    den_o.at[c, pl.ds(s * 640, 640)])


_sc_mv = functools.partial(
    pl.kernel,
    out_type=[
        jax.ShapeDtypeStruct((2, NPAD, F), jnp.float32),
        jax.ShapeDtypeStruct((2, NPAD), jnp.float32),
    ],
    mesh=plsc.VectorSubcoreMesh(core_axis_name="c", subcore_axis_name="s",
                                num_cores=2, num_subcores=16),
    compiler_params=pltpu.CompilerParams(needs_layout_passes=False),
    scratch_types=[
        pltpu.VMEM((NPAD,), jnp.float32),
        pltpu.VMEM((8, 128), jnp.int32),
        pltpu.VMEM((8, 128), jnp.int32),
        pltpu.VMEM((8, 128), jnp.float32),
        pltpu.VMEM((128, F), jnp.float32),
        pltpu.VMEM((128, F), jnp.float32),
        pltpu.VMEM((1, 128), jnp.float32),
        pltpu.VMEM((1, 128), jnp.float32),
        pltpu.VMEM_SHARED((NPAD, F), jnp.float32),
        pltpu.VMEM_SHARED((NPAD,), jnp.float32),
        pltpu.SemaphoreType.DMA,
        pltpu.SemaphoreType.DMA,
    ],
)


def _sc_me_body(recv2, send2, r_hbm, s_hbm, me_o,
                ridx8, sidx8, ba0, bb0, ba1, bb1,
                sa0, sb0, sa1, sb1):
    c = lax.axis_index("c")
    s = lax.axis_index("s")
    nrows = jnp.where(c == 0, RPW0, RPW1)
    base = pl.multiple_of(jnp.where(c == 0, s * RPW0, 16 * RPW0 + s * RPW1), 8)
    ng = nrows // 2

    pltpu.sync_copy(recv2.at[pl.ds(base, 8)], ridx8)
    pltpu.sync_copy(send2.at[pl.ds(base, 8)], sidx8)
    pltpu.async_copy(r_hbm.at[ridx8.at[0]], ba0, sa0)
    pltpu.async_copy(s_hbm.at[sidx8.at[0]], bb0, sb0)
    pltpu.async_copy(r_hbm.at[ridx8.at[1]], ba1, sa1)
    pltpu.async_copy(s_hbm.at[sidx8.at[1]], bb1, sb1)

    def _me(ba, bb):
        @plsc.parallel_loop(0, 128, unroll=4)
        def _me1(e):
            for v in range(8):
                a = ba[e, pl.ds(v * 16, 16)] + bb[e, pl.ds(v * 16, 16)]
                ba[e, pl.ds(v * 16, 16)] = jnp.maximum(a, 0.2 * a)

    def _body(g, _):
        ja = lax.rem(2 * g, 8)
        pltpu.make_async_copy(r_hbm.at[ridx8.at[ja]], ba0, sa0).wait()
        pltpu.make_async_copy(s_hbm.at[sidx8.at[ja]], bb0, sb0).wait()
        _me(ba0, bb0)
        pltpu.sync_copy(ba0, me_o.at[pl.ds((base + 2 * g) * 128, 128)])
        pltpu.make_async_copy(r_hbm.at[ridx8.at[ja]], ba1, sa1).wait()
        pltpu.make_async_copy(s_hbm.at[sidx8.at[ja]], bb1, sb1).wait()
        _me(ba1, bb1)

        @pl.when(jnp.logical_and(lax.rem(g, 4) == 3, g < ng - 1))
        def _():
            nxt = base + lax.div(g + 1, 4) * 8
            pltpu.sync_copy(recv2.at[pl.ds(nxt, 8)], ridx8)
            pltpu.sync_copy(send2.at[pl.ds(nxt, 8)], sidx8)

        pltpu.sync_copy(ba1, me_o.at[pl.ds((base + 2 * g + 1) * 128, 128)])

        @pl.when(g < ng - 1)
        def _():
            jn = lax.rem(2 * g + 2, 8)
            pltpu.async_copy(r_hbm.at[ridx8.at[jn]], ba0, sa0)
            pltpu.async_copy(s_hbm.at[sidx8.at[jn]], bb0, sb0)
            pltpu.async_copy(r_hbm.at[ridx8.at[jn + 1]], ba1, sa1)
            pltpu.async_copy(s_hbm.at[sidx8.at[jn + 1]], bb1, sb1)

        return 0

    lax.fori_loop(0, ng, _body, 0)


_sc_me = functools.partial(
    pl.kernel,
    out_type=jax.ShapeDtypeStruct((EPAD, F), jnp.float32),
    mesh=plsc.VectorSubcoreMesh(core_axis_name="c", subcore_axis_name="s",
                                num_cores=2, num_subcores=16),
    compiler_params=pltpu.CompilerParams(needs_layout_passes=False),
    scratch_types=[
        pltpu.VMEM((8, 128), jnp.int32),
        pltpu.VMEM((8, 128), jnp.int32),
        pltpu.VMEM((128, F), jnp.float32),
        pltpu.VMEM((128, F), jnp.float32),
        pltpu.VMEM((128, F), jnp.float32),
        pltpu.VMEM((128, F), jnp.float32),
        pltpu.SemaphoreType.DMA,
        pltpu.SemaphoreType.DMA,
        pltpu.SemaphoreType.DMA,
        pltpu.SemaphoreType.DMA,
    ],
)


def _sc_edge_call(recv2, send2, hw_p, t1, lr, r_n, s_n):
    mv_part, den_part = _sc_mv(_sc_mv_body)(recv2, send2, hw_p, t1, lr)
    me_pad = _sc_me(_sc_me_body)(recv2, send2, r_n, s_n)
    return me_pad, mv_part, den_part


def _fin_body(mv2, den2, out):
    m = mv2[0] + mv2[1]
    d = den2[0, :, :1] + den2[1, :, :1]
    x = m / (d + 1e-9)
    out[...] = jnp.where(x > 0, x, jnp.exp(jnp.minimum(x, 0.0)) - 1.0)


def _tc_finalize(mv_part, den_part):
    nb = 1000
    return pl.pallas_call(
        _fin_body,
        grid=(N // nb,),
        in_specs=[
            pl.BlockSpec((2, nb, F), lambda i: (0, i, 0)),
            pl.BlockSpec((2, nb, 1), lambda i: (0, i, 0)),
        ],
        out_specs=pl.BlockSpec((nb, F), lambda i: (i, 0)),
        out_shape=jax.ShapeDtypeStruct((N, F), jnp.float32),
    )(mv_part, den_part)


def kernel(hv_ftr, he_ftr, p_ftr, q_ftr, edge_index,
           W_attend, b_attend, W_align, b_align, W_e, b_e):
    ei = edge_index.astype(jnp.int32)
    recv, send = ei[0], ei[1]

    wp, wq, whe = W_align[:F], W_align[F:2 * F], W_align[2 * F:]
    We1, We2, We3, We4 = (W_e[:F], W_e[F:2 * F], W_e[2 * F:3 * F], W_e[3 * F:])
    he2 = he_ftr.reshape(E // 8, 128)
    w16 = jnp.kron(jnp.eye(8, dtype=jnp.float32), whe)

    lr, r_n, s_n, t2, hw2 = _tc_precompute(
        hv_ftr, p_ftr, q_ftr, he2, W_attend, b_attend,
        We1, We2, We3, We4, b_e, wp, wq, w16, b_align)

    pad = EPAD - E
    zi = (jnp.arange(pad, dtype=jnp.int32) * 131) % N
    recv2 = jnp.concatenate([recv, zi]).reshape(ROWS_PAD, 128)
    send2 = jnp.concatenate([send, zi]).reshape(ROWS_PAD, 128)
    hw_p = jnp.concatenate([hw2.reshape(E), jnp.full((pad,), -1e30, jnp.float32)]
                           ).reshape(ROWS_PAD, 128)

    me_pad, mv_part, den_part = _sc_edge_call(
        recv2, send2, hw_p, t2[:, 0], lr, r_n, s_n)

    mv_ftr = _tc_finalize(mv_part[:, :N], den_part[:, :N, None])
    return mv_ftr, me_pad[:E]
